# baseline (speedup 1.0000x reference)
import functools

import jax
import jax.numpy as jnp
from jax import lax
from jax.experimental import pallas as pl
from jax.experimental.pallas import tpu as pltpu
from jax.experimental.pallas import tpu_sc as plsc

D = 64
SCALE = 8.0
NBUF = 4
BB = 128


@functools.lru_cache(maxsize=None)
def _make_sc_kernel(nbatch: int, seq: int, vocab: int):
    info = plsc.get_sparse_core_info()
    nw = info.num_cores * info.num_subcores
    n_units = seq * (nbatch // BB)
    u_per_w = n_units // nw
    nquads = u_per_w // NBUF
    ncols = nbatch // BB
    mesh = plsc.VectorSubcoreMesh(core_axis_name="c", subcore_axis_name="s")

    @functools.partial(
        pl.kernel,
        mesh=mesh,
        out_type=jax.ShapeDtypeStruct((seq, D // 8, ncols, 8, BB), jnp.float32),
        compiler_params=pltpu.CompilerParams(
            use_tc_tiling_on_sc=False, needs_layout_passes=False
        ),
        scratch_types=[
            [pltpu.VMEM((BB,), jnp.int32) for _ in range(NBUF)],
            [pltpu.VMEM((BB, D), jnp.float32) for _ in range(NBUF)],
            [pltpu.VMEM((D // 8, 8, BB + 8), jnp.float32) for _ in range(NBUF)],
            [pltpu.SemaphoreType.DMA for _ in range(NBUF)],
            [pltpu.SemaphoreType.DMA for _ in range(NBUF)],
            [pltpu.SemaphoreType.DMA for _ in range(NBUF)],
        ],
    )
    def k(idx_hbm, t2_hbm, out_hbm, idxs, rows, outs, isems, gsems, osems):
        wid = lax.axis_index("s") * info.num_cores + lax.axis_index("c")
        ubase = wid * u_per_w

        def unit_sc(u):
            g = ubase + u
            return lax.div(g, ncols), lax.rem(g, ncols)

        def idx_copy(u, b):
            s, c = unit_sc(u)
            src = idx_hbm.at[s, pl.ds(c * BB, BB)]
            return pltpu.make_async_copy(src, idxs[b], isems[b])

        def gather_copy(b):
            return pltpu.make_async_copy(t2_hbm.at[idxs[b]], rows[b], gsems[b])

        def out_copy(u, b):
            s, c = unit_sc(u)
            src = outs[b].at[:, :, pl.ds(0, BB)]
            dst = out_hbm.at[s, :, c, :, :]
            return pltpu.make_async_copy(src, dst, osems[b])

        def transform(b):
            rows_b, outb = rows[b], outs[b]
            iota = lax.iota(jnp.int32, 16)

            def d_body(t, car):
                dv = t * 16 + iota
                rv = lax.shift_right_logical(dv, 3)
                rrv = lax.bitwise_and(dv, 7)

                @plsc.parallel_loop(0, BB, unroll=8)
                def _(c):
                    v = rows_b[c, pl.ds(t * 16, 16)] * SCALE
                    cv = jnp.full((16,), c, jnp.int32)
                    plsc.store_scatter(outb, [rv, rrv, cv], v)

                return car

            lax.fori_loop(0, D // 16, d_body, 0)

        for b in range(NBUF):
            idx_copy(b, b).start()
        for b in range(NBUF):
            idx_copy(b, b).wait()
            gather_copy(b).start()

        def body(q, carry):
            u0 = q * NBUF
            for b in range(NBUF):
                u1 = u0 + NBUF + b

                @pl.when(u1 < u_per_w)
                def _():
                    idx_copy(u1, b).start()

            for b in range(NBUF):
                u = u0 + b

                @pl.when(q > 0)
                def _():
                    out_copy(u - NBUF, b).wait()

                gather_copy(b).wait()
                transform(b)
                out_copy(u, b).start()
            for b in range(NBUF):
                u1 = u0 + NBUF + b

                @pl.when(u1 < u_per_w)
                def _():
                    idx_copy(u1, b).wait()
                    gather_copy(b).start()

            return carry

        lax.fori_loop(0, nquads, body, 0)

        u0 = (nquads - 1) * NBUF
        for b in range(NBUF):
            out_copy(u0 + b, b).wait()

    return k


def kernel(xb, table):
    nb, seq = xb.shape
    vocab = table.shape[0]
    xbT = xb.T.astype(jnp.int32)
    a = _make_sc_kernel(nb, seq, vocab)(xbT, table)
    return a.transpose((2, 4, 0, 1, 3)).reshape(nb, seq, D)


# speedup vs baseline: 1.0040x; 1.0040x over previous
import functools

import jax
import jax.numpy as jnp
from jax import lax
from jax.experimental import pallas as pl
from jax.experimental.pallas import tpu as pltpu
from jax.experimental.pallas import tpu_sc as plsc

D = 64
SCALE = 8.0
NBUF = 4
BB = 128


@functools.lru_cache(maxsize=None)
def _make_sc_kernel(nbatch: int, seq: int, vocab: int):
    info = plsc.get_sparse_core_info()
    nw = info.num_cores * info.num_subcores
    n_units = seq * (nbatch // BB)
    u_per_w = n_units // nw
    nquads = u_per_w // NBUF
    ncols = nbatch // BB
    mesh = plsc.VectorSubcoreMesh(core_axis_name="c", subcore_axis_name="s")

    @functools.partial(
        pl.kernel,
        mesh=mesh,
        out_type=jax.ShapeDtypeStruct((seq, D // 8, ncols, 8, BB), jnp.float32),
        compiler_params=pltpu.CompilerParams(
            use_tc_tiling_on_sc=False, needs_layout_passes=False
        ),
        scratch_types=[
            [pltpu.VMEM((BB,), jnp.int32) for _ in range(NBUF)],
            [pltpu.VMEM((BB, D), jnp.float32) for _ in range(NBUF)],
            [pltpu.VMEM((D // 8, 8, BB + 8), jnp.float32) for _ in range(NBUF)],
            [pltpu.SemaphoreType.DMA for _ in range(NBUF)],
            [pltpu.SemaphoreType.DMA for _ in range(NBUF)],
            [pltpu.SemaphoreType.DMA for _ in range(NBUF)],
        ],
    )
    def k(idx_hbm, t2_hbm, out_hbm, idxs, rows, outs, isems, gsems, osems):
        wid = lax.axis_index("s") * info.num_cores + lax.axis_index("c")
        ubase = wid * u_per_w

        def unit_sc(u):
            g = ubase + u
            return lax.div(g, ncols), lax.rem(g, ncols)

        def idx_copy(u, b):
            s, c = unit_sc(u)
            src = idx_hbm.at[s, pl.ds(c * BB, BB)]
            return pltpu.make_async_copy(src, idxs[b], isems[b])

        def gather_copy(b):
            return pltpu.make_async_copy(t2_hbm.at[idxs[b]], rows[b], gsems[b])

        def out_copy(u, b):
            s, c = unit_sc(u)
            src = outs[b].at[:, :, pl.ds(0, BB)]
            dst = out_hbm.at[s, :, c, :, :]
            return pltpu.make_async_copy(src, dst, osems[b])

        def transform(b):
            rows_b, outb = rows[b], outs[b]
            iota = lax.iota(jnp.int32, 16)

            def d_body(t, car):
                dv = t * 16 + iota
                rv = lax.shift_right_logical(dv, 3)
                rrv = lax.bitwise_and(dv, 7)

                @plsc.parallel_loop(0, BB, unroll=4)
                def _(c):
                    v = rows_b[c, pl.ds(t * 16, 16)] * SCALE
                    cv = jnp.full((16,), c, jnp.int32)
                    plsc.store_scatter(outb, [rv, rrv, cv], v)

                return car

            lax.fori_loop(0, D // 16, d_body, 0)

        for b in range(NBUF):
            idx_copy(b, b).start()
        for b in range(NBUF):
            idx_copy(b, b).wait()
            gather_copy(b).start()

        def body(q, carry):
            u0 = q * NBUF
            for b in range(NBUF):
                u1 = u0 + NBUF + b

                @pl.when(u1 < u_per_w)
                def _():
                    idx_copy(u1, b).start()

            for b in range(NBUF):
                u = u0 + b

                @pl.when(q > 0)
                def _():
                    out_copy(u - NBUF, b).wait()

                gather_copy(b).wait()
                transform(b)
                out_copy(u, b).start()
            for b in range(NBUF):
                u1 = u0 + NBUF + b

                @pl.when(u1 < u_per_w)
                def _():
                    idx_copy(u1, b).wait()
                    gather_copy(b).start()

            return carry

        lax.fori_loop(0, nquads, body, 0)

        u0 = (nquads - 1) * NBUF
        for b in range(NBUF):
            out_copy(u0 + b, b).wait()

    return k


def kernel(xb, table):
    nb, seq = xb.shape
    vocab = table.shape[0]
    xbT = xb.T.astype(jnp.int32)
    a = _make_sc_kernel(nb, seq, vocab)(xbT, table)
    return a.transpose((2, 4, 0, 1, 3)).reshape(nb, seq, D)
